# baseline (speedup 1.0000x reference)
import functools

import jax
import jax.numpy as jnp
from jax import lax
from jax.experimental import pallas as pl
from jax.experimental.pallas import tpu as pltpu
from jax.experimental.pallas import tpu_sc as plsc

B = 4
N = 4096
M = 4096
K = 32
L = 16
ROWS_PER_W = (B * M) // 32
CHUNKS = M // ROWS_PER_W
VPB = 4
BLK = VPB * L
BUF = 32
FLUSH_AT = L

_INF = float("inf")


def _sort16(k, v):
    return plsc.sort_key_val(k, v)


def _merge16(ak, ai, bk, bi):
    rk = lax.rev(bk, (0,))
    ri = lax.rev(bi, (0,))
    m = ak <= rk
    lok = jnp.where(m, ak, rk)
    loi = jnp.where(m, ai, ri)
    hik = jnp.where(m, rk, ak)
    hii = jnp.where(m, ri, ai)
    o0k, o0i = _sort16(lok, loi)
    o1k, o1i = _sort16(hik, hii)
    return o0k, o0i, o1k, o1i


def _low32(a0k, a0i, a1k, a1i, b0k, b0i, b1k, b1i):
    rb0k = lax.rev(b1k, (0,))
    rb0i = lax.rev(b1i, (0,))
    rb1k = lax.rev(b0k, (0,))
    rb1i = lax.rev(b0i, (0,))
    m0 = a0k <= rb0k
    c0k = jnp.where(m0, a0k, rb0k)
    c0i = jnp.where(m0, a0i, rb0i)
    m1 = a1k <= rb1k
    c1k = jnp.where(m1, a1k, rb1k)
    c1i = jnp.where(m1, a1i, rb1i)
    m = c0k <= c1k
    lk = jnp.where(m, c0k, c1k)
    li = jnp.where(m, c0i, c1i)
    hk = jnp.where(m, c1k, c0k)
    hi = jnp.where(m, c1i, c0i)
    o0k, o0i = _sort16(lk, li)
    o1k, o1i = _sort16(hk, hi)
    return o0k, o0i, o1k, o1i


def _knn_body(x1_hbm, x2_hbm, out_hbm, cx, cy, cz, qx, qy, qz,
              bufk, bufi, outv):
    info = plsc.get_sparse_core_info()
    nc = info.num_cores
    wid = lax.axis_index("s") * nc + lax.axis_index("c")
    b = wid // CHUNKS
    chunk = wid % CHUNKS
    row0 = chunk * ROWS_PER_W

    pltpu.sync_copy(x1_hbm.at[b * 3 + 0], cx)
    pltpu.sync_copy(x1_hbm.at[b * 3 + 1], cy)
    pltpu.sync_copy(x1_hbm.at[b * 3 + 2], cz)
    pltpu.sync_copy(x2_hbm.at[b * 3 + 0, pl.ds(row0, ROWS_PER_W)], qx)
    pltpu.sync_copy(x2_hbm.at[b * 3 + 1, pl.ds(row0, ROWS_PER_W)], qy)
    pltpu.sync_copy(x2_hbm.at[b * 3 + 2, pl.ds(row0, ROWS_PER_W)], qz)

    iota = lax.iota(jnp.int32, L)
    inf_vec = jnp.full((L,), _INF, jnp.float32)
    zero_vec = jnp.zeros((L,), jnp.int32)

    def flush(cnt, r0k, r0i, r1k, r1i):
        s0k, s0i = _sort16(bufk[pl.ds(0, L)], bufi[pl.ds(0, L)])
        s1k, s1i = _sort16(bufk[pl.ds(L, L)], bufi[pl.ds(L, L)])
        bk0, bi0, bk1, bi1 = _merge16(s0k, s0i, s1k, s1i)
        n0k, n0i, n1k, n1i = _low32(bk0, bi0, bk1, bi1, r0k, r0i, r1k, r1i)
        bufk[pl.ds(0, L)] = inf_vec
        bufk[pl.ds(L, L)] = inf_vec
        t = jnp.full((L,), n1k[15], jnp.float32)
        return t, jnp.int32(0), n0k, n0i, n1k, n1i

    def row_body(m, _):
        bufk[pl.ds(0, L)] = inf_vec
        bufk[pl.ds(L, L)] = inf_vec

        idxm = jnp.full((L,), m, jnp.int32)
        qxv = plsc.load_gather(qx, [idxm])
        qyv = plsc.load_gather(qy, [idxm])
        qzv = plsc.load_gather(qz, [idxm])

        def dist(base_u):
            dx = cx[pl.ds(base_u, L)] - qxv
            dy = cy[pl.ds(base_u, L)] - qyv
            dz = cz[pl.ds(base_u, L)] - qzv
            return dx * dx + dy * dy + dz * dz

        def blk_body(j, carry):
            t, cnt, r0k, r0i, r1k, r1i = carry
            base = j * BLK
            ds = [dist(base + u * L) for u in range(VPB)]
            ks = [d <= t for d in ds]
            anyk = ks[0]
            for u in range(1, VPB):
                anyk = anyk | ks[u]
            hit = jnp.any(anyk)

            def slow(t, cnt, r0k, r0i, r1k, r1i, *ds):
                for u in range(VPB):
                    k = ds[u] <= t
                    pc = plsc.all_reduce_population_count(k)[0]
                    plsc.store_compressed(bufk.at[pl.ds(cnt, L)], ds[u],
                                          mask=k)
                    plsc.store_compressed(bufi.at[pl.ds(cnt, L)],
                                          iota + (base + u * L), mask=k)
                    cnt = cnt + pc
                    t, cnt, r0k, r0i, r1k, r1i = lax.cond(
                        cnt >= FLUSH_AT, flush,
                        lambda c, a, b, d, e: (t, c, a, b, d, e),
                        cnt, r0k, r0i, r1k, r1i)
                return t, cnt, r0k, r0i, r1k, r1i

            def fast(t, cnt, r0k, r0i, r1k, r1i, *ds):
                return t, cnt, r0k, r0i, r1k, r1i

            return lax.cond(hit, slow, fast, t, cnt, r0k, r0i, r1k, r1i, *ds)

        init = (inf_vec, jnp.int32(0), inf_vec, zero_vec, inf_vec, zero_vec)
        t, cnt, r0k, r0i, r1k, r1i = lax.fori_loop(
            0, N // BLK, blk_body, init)
        _, _, r0k, r0i, r1k, r1i = flush(cnt, r0k, r0i, r1k, r1i)
        outv[pl.ds(m * K, L)] = r0i
        outv[pl.ds(m * K + L, L)] = r1i
        return 0

    lax.fori_loop(0, ROWS_PER_W, row_body, 0)
    pltpu.sync_copy(outv, out_hbm.at[pl.ds((b * M + row0) * K,
                                           ROWS_PER_W * K)])


@jax.jit
def _knn_sc(x1t, x2t):
    mesh = plsc.VectorSubcoreMesh(core_axis_name="c", subcore_axis_name="s")
    f = functools.partial(
        pl.kernel,
        out_type=jax.ShapeDtypeStruct((B * M * K,), jnp.int32),
        mesh=mesh,
        compiler_params=pltpu.CompilerParams(needs_layout_passes=False),
        scratch_types=[
            pltpu.VMEM((N,), jnp.float32),
            pltpu.VMEM((N,), jnp.float32),
            pltpu.VMEM((N,), jnp.float32),
            pltpu.VMEM((ROWS_PER_W,), jnp.float32),
            pltpu.VMEM((ROWS_PER_W,), jnp.float32),
            pltpu.VMEM((ROWS_PER_W,), jnp.float32),
            pltpu.VMEM((BUF,), jnp.float32),
            pltpu.VMEM((BUF,), jnp.int32),
            pltpu.VMEM((ROWS_PER_W * K,), jnp.int32),
        ],
    )(_knn_body)
    return f(x1t, x2t)


def kernel(xyz1, xyz2):
    x1t = xyz1.transpose(0, 2, 1).reshape(B * 3, N)
    x2t = xyz2.transpose(0, 2, 1).reshape(B * 3, M)
    out = _knn_sc(x1t, x2t)
    return out.reshape(B, M, K, 1)


# speedup vs baseline: 3.0025x; 3.0025x over previous
import functools

import jax
import jax.numpy as jnp
from jax import lax
from jax.experimental import pallas as pl
from jax.experimental.pallas import tpu as pltpu
from jax.experimental.pallas import tpu_sc as plsc

B = 4
N = 4096
M = 4096
K = 32
L = 16
ROWS_PER_W = (B * M) // 32
CHUNKS = M // ROWS_PER_W

_INF = float("inf")


def _sort16(k, v):
    return plsc.sort_key_val(k, v)


def _merge16(ak, ai, bk, bi):
    rk = lax.rev(bk, (0,))
    ri = lax.rev(bi, (0,))
    m = ak <= rk
    lok = jnp.where(m, ak, rk)
    loi = jnp.where(m, ai, ri)
    hik = jnp.where(m, rk, ak)
    hii = jnp.where(m, ri, ai)
    o0k, o0i = _sort16(lok, loi)
    o1k, o1i = _sort16(hik, hii)
    return o0k, o0i, o1k, o1i


def _fold16(sk, si, r0k, r0i, r1k, r1i):
    rsk = lax.rev(sk, (0,))
    rsi = lax.rev(si, (0,))
    m1 = r1k <= rsk
    c1k = jnp.where(m1, r1k, rsk)
    c1i = jnp.where(m1, r1i, rsi)
    m = r0k <= c1k
    lk = jnp.where(m, r0k, c1k)
    li = jnp.where(m, r0i, c1i)
    hk = jnp.where(m, c1k, r0k)
    hi = jnp.where(m, c1i, r0i)
    o0k, o0i = _sort16(lk, li)
    o1k, o1i = _sort16(hk, hi)
    return o0k, o0i, o1k, o1i


def _knn_body(x1_hbm, x2_hbm, out_hbm, cx, cy, cz, qx, qy, qz,
              dbuf, bufk, bufi, outv):
    info = plsc.get_sparse_core_info()
    nc = info.num_cores
    wid = lax.axis_index("s") * nc + lax.axis_index("c")
    b = wid // CHUNKS
    chunk = wid % CHUNKS
    row0 = chunk * ROWS_PER_W

    pltpu.sync_copy(x1_hbm.at[b * 3 + 0], cx)
    pltpu.sync_copy(x1_hbm.at[b * 3 + 1], cy)
    pltpu.sync_copy(x1_hbm.at[b * 3 + 2], cz)
    pltpu.sync_copy(x2_hbm.at[b * 3 + 0, pl.ds(row0, ROWS_PER_W)], qx)
    pltpu.sync_copy(x2_hbm.at[b * 3 + 1, pl.ds(row0, ROWS_PER_W)], qy)
    pltpu.sync_copy(x2_hbm.at[b * 3 + 2, pl.ds(row0, ROWS_PER_W)], qz)

    iota = lax.iota(jnp.int32, L)
    inf_vec = jnp.full((L,), _INF, jnp.float32)
    zero_vec = jnp.zeros((L,), jnp.int32)

    def row_body(m, _):
        idxm = jnp.full((L,), m, jnp.int32)
        qxv = plsc.load_gather(qx, [idxm])
        qyv = plsc.load_gather(qy, [idxm])
        qzv = plsc.load_gather(qz, [idxm])

        def pa(j, carry):
            mn0, mn1 = carry
            base = j * 2 * L
            dx0 = cx[pl.ds(base, L)] - qxv
            dy0 = cy[pl.ds(base, L)] - qyv
            dz0 = cz[pl.ds(base, L)] - qzv
            d0 = dx0 * dx0 + dy0 * dy0 + dz0 * dz0
            dx1 = cx[pl.ds(base + L, L)] - qxv
            dy1 = cy[pl.ds(base + L, L)] - qyv
            dz1 = cz[pl.ds(base + L, L)] - qzv
            d1 = dx1 * dx1 + dy1 * dy1 + dz1 * dz1
            dbuf[pl.ds(base, L)] = d0
            dbuf[pl.ds(base + L, L)] = d1
            return jnp.minimum(mn0, d0), jnp.minimum(mn1, d1)

        mn0, mn1 = lax.fori_loop(0, N // (2 * L), pa, (inf_vec, inf_vec),
                                 unroll=2)
        t0 = jnp.max(jnp.maximum(mn0, mn1))
        t0v = jnp.full((L,), t0, jnp.float32)

        def pb(j, cnt):
            dv = dbuf[pl.ds(j * L, L)]
            k = dv <= t0v
            pc = plsc.all_reduce_population_count(k)[0]
            plsc.store_compressed(bufk.at[pl.ds(cnt, L)], dv, mask=k)
            plsc.store_compressed(bufi.at[pl.ds(cnt, L)], iota + j * L,
                                  mask=k)
            return cnt + pc

        cnt = lax.fori_loop(0, N // L, pb, jnp.int32(0), unroll=4)
        bufk[pl.ds(cnt, L)] = inf_vec

        s0k, s0i = _sort16(bufk[pl.ds(0, L)], bufi[pl.ds(0, L)])
        s1k, s1i = _sort16(bufk[pl.ds(L, L)], bufi[pl.ds(L, L)])
        r = _merge16(s0k, s0i, s1k, s1i)

        def pc_body(v, carry):
            sk, si = _sort16(bufk[pl.ds(v * L, L)], bufi[pl.ds(v * L, L)])
            return _fold16(sk, si, *carry)

        nb = (cnt + (L - 1)) // L
        r0k, r0i, r1k, r1i = lax.fori_loop(2, nb, pc_body, r)
        outv[pl.ds(m * K, L)] = r0i
        outv[pl.ds(m * K + L, L)] = r1i
        return 0

    lax.fori_loop(0, ROWS_PER_W, row_body, 0)
    pltpu.sync_copy(outv, out_hbm.at[pl.ds((b * M + row0) * K,
                                           ROWS_PER_W * K)])


@jax.jit
def _knn_sc(x1t, x2t):
    mesh = plsc.VectorSubcoreMesh(core_axis_name="c", subcore_axis_name="s")
    f = functools.partial(
        pl.kernel,
        out_type=jax.ShapeDtypeStruct((B * M * K,), jnp.int32),
        mesh=mesh,
        compiler_params=pltpu.CompilerParams(needs_layout_passes=False),
        scratch_types=[
            pltpu.VMEM((N,), jnp.float32),
            pltpu.VMEM((N,), jnp.float32),
            pltpu.VMEM((N,), jnp.float32),
            pltpu.VMEM((ROWS_PER_W,), jnp.float32),
            pltpu.VMEM((ROWS_PER_W,), jnp.float32),
            pltpu.VMEM((ROWS_PER_W,), jnp.float32),
            pltpu.VMEM((N,), jnp.float32),
            pltpu.VMEM((N + 2 * L,), jnp.float32),
            pltpu.VMEM((N + 2 * L,), jnp.int32),
            pltpu.VMEM((ROWS_PER_W * K,), jnp.int32),
        ],
    )(_knn_body)
    return f(x1t, x2t)


def kernel(xyz1, xyz2):
    x1t = xyz1.transpose(0, 2, 1).reshape(B * 3, N)
    x2t = xyz2.transpose(0, 2, 1).reshape(B * 3, M)
    out = _knn_sc(x1t, x2t)
    return out.reshape(B, M, K, 1)


# speedup vs baseline: 10.7300x; 3.5737x over previous
import functools

import jax
import jax.numpy as jnp
from jax import lax
from jax.experimental import pallas as pl
from jax.experimental.pallas import tpu as pltpu
from jax.experimental.pallas import tpu_sc as plsc

B = 4
N = 4096
M = 4096
K = 32
L = 16
ROWS_PER_W = (B * M) // 32
CHUNKS = M // ROWS_PER_W

_INF = float("inf")


def _sort16(k, v):
    return plsc.sort_key_val(k, v)


def _merge16(ak, ai, bk, bi):
    rk = lax.rev(bk, (0,))
    ri = lax.rev(bi, (0,))
    m = ak <= rk
    lok = jnp.where(m, ak, rk)
    loi = jnp.where(m, ai, ri)
    hik = jnp.where(m, rk, ak)
    hii = jnp.where(m, ri, ai)
    o0k, o0i = _sort16(lok, loi)
    o1k, o1i = _sort16(hik, hii)
    return o0k, o0i, o1k, o1i


def _fold16(sk, si, r0k, r0i, r1k, r1i):
    rsk = lax.rev(sk, (0,))
    rsi = lax.rev(si, (0,))
    m1 = r1k <= rsk
    c1k = jnp.where(m1, r1k, rsk)
    c1i = jnp.where(m1, r1i, rsi)
    m = r0k <= c1k
    lk = jnp.where(m, r0k, c1k)
    li = jnp.where(m, r0i, c1i)
    hk = jnp.where(m, c1k, r0k)
    hi = jnp.where(m, c1i, r0i)
    o0k, o0i = _sort16(lk, li)
    o1k, o1i = _sort16(hk, hi)
    return o0k, o0i, o1k, o1i


def _knn_body(x1_hbm, x2_hbm, out_hbm, cx, cy, cz, qx, qy, qz,
              dbuf, bufk, bufi, outv):
    info = plsc.get_sparse_core_info()
    nc = info.num_cores
    wid = lax.axis_index("s") * nc + lax.axis_index("c")
    b = wid // CHUNKS
    chunk = wid % CHUNKS
    row0 = chunk * ROWS_PER_W

    pltpu.sync_copy(x1_hbm.at[b * 3 + 0], cx)
    pltpu.sync_copy(x1_hbm.at[b * 3 + 1], cy)
    pltpu.sync_copy(x1_hbm.at[b * 3 + 2], cz)
    pltpu.sync_copy(x2_hbm.at[b * 3 + 0, pl.ds(row0, ROWS_PER_W)], qx)
    pltpu.sync_copy(x2_hbm.at[b * 3 + 1, pl.ds(row0, ROWS_PER_W)], qy)
    pltpu.sync_copy(x2_hbm.at[b * 3 + 2, pl.ds(row0, ROWS_PER_W)], qz)

    iota = lax.iota(jnp.int32, L)
    inf_vec = jnp.full((L,), _INF, jnp.float32)
    zero_vec = jnp.zeros((L,), jnp.int32)

    def row_body(m, _):
        idxm = jnp.full((L,), m, jnp.int32)
        qxv = plsc.load_gather(qx, [idxm])
        qyv = plsc.load_gather(qy, [idxm])
        qzv = plsc.load_gather(qz, [idxm])

        def pa(j, carry):
            mn0, mn1 = carry
            base = j * 2 * L
            dx0 = cx[pl.ds(base, L)] - qxv
            dy0 = cy[pl.ds(base, L)] - qyv
            dz0 = cz[pl.ds(base, L)] - qzv
            d0 = dx0 * dx0 + dy0 * dy0 + dz0 * dz0
            dx1 = cx[pl.ds(base + L, L)] - qxv
            dy1 = cy[pl.ds(base + L, L)] - qyv
            dz1 = cz[pl.ds(base + L, L)] - qzv
            d1 = dx1 * dx1 + dy1 * dy1 + dz1 * dz1
            dbuf[pl.ds(base, L)] = d0
            dbuf[pl.ds(base + L, L)] = d1
            return jnp.minimum(mn0, d0), jnp.minimum(mn1, d1)

        mn0, mn1 = plsc.parallel_loop(
            0, N // (2 * L), carry=(inf_vec, inf_vec), unroll=4)(pa)
        t0 = jnp.max(jnp.maximum(mn0, mn1))
        t0v = jnp.full((L,), t0, jnp.float32)

        def pb(j, cnt):
            dv = dbuf[pl.ds(j * L, L)]
            k = dv <= t0v
            pc = plsc.all_reduce_population_count(k)[0]
            plsc.store_compressed(bufk.at[pl.ds(cnt, L)], dv, mask=k)
            plsc.store_compressed(bufi.at[pl.ds(cnt, L)], iota + j * L,
                                  mask=k)
            return cnt + pc

        cnt = plsc.parallel_loop(
            0, N // L, carry=jnp.int32(0), unroll=4)(pb)
        bufk[pl.ds(cnt, L)] = inf_vec

        s0k, s0i = _sort16(bufk[pl.ds(0, L)], bufi[pl.ds(0, L)])
        s1k, s1i = _sort16(bufk[pl.ds(L, L)], bufi[pl.ds(L, L)])
        r = _merge16(s0k, s0i, s1k, s1i)

        def pc_body(v, carry):
            sk, si = _sort16(bufk[pl.ds(v * L, L)], bufi[pl.ds(v * L, L)])
            return _fold16(sk, si, *carry)

        nb = (cnt + (L - 1)) // L
        r0k, r0i, r1k, r1i = lax.fori_loop(2, nb, pc_body, r)
        outv[pl.ds(m * K, L)] = r0i
        outv[pl.ds(m * K + L, L)] = r1i
        return 0

    lax.fori_loop(0, ROWS_PER_W, row_body, 0)
    pltpu.sync_copy(outv, out_hbm.at[pl.ds((b * M + row0) * K,
                                           ROWS_PER_W * K)])


@jax.jit
def _knn_sc(x1t, x2t):
    mesh = plsc.VectorSubcoreMesh(core_axis_name="c", subcore_axis_name="s")
    f = functools.partial(
        pl.kernel,
        out_type=jax.ShapeDtypeStruct((B * M * K,), jnp.int32),
        mesh=mesh,
        compiler_params=pltpu.CompilerParams(needs_layout_passes=False),
        scratch_types=[
            pltpu.VMEM((N,), jnp.float32),
            pltpu.VMEM((N,), jnp.float32),
            pltpu.VMEM((N,), jnp.float32),
            pltpu.VMEM((ROWS_PER_W,), jnp.float32),
            pltpu.VMEM((ROWS_PER_W,), jnp.float32),
            pltpu.VMEM((ROWS_PER_W,), jnp.float32),
            pltpu.VMEM((N,), jnp.float32),
            pltpu.VMEM((N + 2 * L,), jnp.float32),
            pltpu.VMEM((N + 2 * L,), jnp.int32),
            pltpu.VMEM((ROWS_PER_W * K,), jnp.int32),
        ],
    )(_knn_body)
    return f(x1t, x2t)


def kernel(xyz1, xyz2):
    x1t = xyz1.transpose(0, 2, 1).reshape(B * 3, N)
    x2t = xyz2.transpose(0, 2, 1).reshape(B * 3, M)
    out = _knn_sc(x1t, x2t)
    return out.reshape(B, M, K, 1)


# speedup vs baseline: 12.4932x; 1.1643x over previous
import functools

import jax
import jax.numpy as jnp
from jax import lax
from jax.experimental import pallas as pl
from jax.experimental.pallas import tpu as pltpu
from jax.experimental.pallas import tpu_sc as plsc

B = 4
N = 4096
M = 4096
K = 32
L = 16
ROWS_PER_W = (B * M) // 32
CHUNKS = M // ROWS_PER_W

_INF = float("inf")


def _sort16(k, v):
    return plsc.sort_key_val(k, v)


def _merge16(ak, ai, bk, bi):
    rk = lax.rev(bk, (0,))
    ri = lax.rev(bi, (0,))
    m = ak <= rk
    lok = jnp.where(m, ak, rk)
    loi = jnp.where(m, ai, ri)
    hik = jnp.where(m, rk, ak)
    hii = jnp.where(m, ri, ai)
    o0k, o0i = _sort16(lok, loi)
    o1k, o1i = _sort16(hik, hii)
    return o0k, o0i, o1k, o1i


def _low32(a0k, a0i, a1k, a1i, b0k, b0i, b1k, b1i):
    rb0k = lax.rev(b1k, (0,))
    rb0i = lax.rev(b1i, (0,))
    rb1k = lax.rev(b0k, (0,))
    rb1i = lax.rev(b0i, (0,))
    m0 = a0k <= rb0k
    c0k = jnp.where(m0, a0k, rb0k)
    c0i = jnp.where(m0, a0i, rb0i)
    m1 = a1k <= rb1k
    c1k = jnp.where(m1, a1k, rb1k)
    c1i = jnp.where(m1, a1i, rb1i)
    m = c0k <= c1k
    lk = jnp.where(m, c0k, c1k)
    li = jnp.where(m, c0i, c1i)
    hk = jnp.where(m, c1k, c0k)
    hi = jnp.where(m, c1i, c0i)
    o0k, o0i = _sort16(lk, li)
    o1k, o1i = _sort16(hk, hi)
    return o0k, o0i, o1k, o1i


def _fold16(sk, si, r0k, r0i, r1k, r1i):
    rsk = lax.rev(sk, (0,))
    rsi = lax.rev(si, (0,))
    m1 = r1k <= rsk
    c1k = jnp.where(m1, r1k, rsk)
    c1i = jnp.where(m1, r1i, rsi)
    m = r0k <= c1k
    lk = jnp.where(m, r0k, c1k)
    li = jnp.where(m, r0i, c1i)
    hk = jnp.where(m, c1k, r0k)
    hi = jnp.where(m, c1i, r0i)
    o0k, o0i = _sort16(lk, li)
    o1k, o1i = _sort16(hk, hi)
    return o0k, o0i, o1k, o1i


def _knn_body(x1_hbm, x2_hbm, out_hbm, cx, cy, cz, qx, qy, qz,
              dbuf, bufi, outv):
    info = plsc.get_sparse_core_info()
    nc = info.num_cores
    wid = lax.axis_index("s") * nc + lax.axis_index("c")
    b = wid // CHUNKS
    chunk = wid % CHUNKS
    row0 = chunk * ROWS_PER_W

    pltpu.sync_copy(x1_hbm.at[b * 3 + 0], cx)
    pltpu.sync_copy(x1_hbm.at[b * 3 + 1], cy)
    pltpu.sync_copy(x1_hbm.at[b * 3 + 2], cz)
    pltpu.sync_copy(x2_hbm.at[b * 3 + 0, pl.ds(row0, ROWS_PER_W)], qx)
    pltpu.sync_copy(x2_hbm.at[b * 3 + 1, pl.ds(row0, ROWS_PER_W)], qy)
    pltpu.sync_copy(x2_hbm.at[b * 3 + 2, pl.ds(row0, ROWS_PER_W)], qz)

    iota = lax.iota(jnp.int32, L)
    inf_vec = jnp.full((L,), _INF, jnp.float32)
    zero_vec = jnp.zeros((L,), jnp.int32)

    dbuf[pl.ds(N, L)] = jnp.full((L,), _INF, jnp.float32)

    def row_body(m, _):
        idxm = jnp.full((L,), m, jnp.int32)
        qxv = plsc.load_gather(qx, [idxm])
        qyv = plsc.load_gather(qy, [idxm])
        qzv = plsc.load_gather(qz, [idxm])

        def pa(j, carry):
            mn0, mn1 = carry
            base = j * 2 * L
            dx0 = cx[pl.ds(base, L)] - qxv
            dy0 = cy[pl.ds(base, L)] - qyv
            dz0 = cz[pl.ds(base, L)] - qzv
            d0 = dx0 * dx0 + dy0 * dy0 + dz0 * dz0
            dx1 = cx[pl.ds(base + L, L)] - qxv
            dy1 = cy[pl.ds(base + L, L)] - qyv
            dz1 = cz[pl.ds(base + L, L)] - qzv
            d1 = dx1 * dx1 + dy1 * dy1 + dz1 * dz1
            dbuf[pl.ds(base, L)] = d0
            dbuf[pl.ds(base + L, L)] = d1
            return jnp.minimum(mn0, d0), jnp.minimum(mn1, d1)

        mn0, mn1 = plsc.parallel_loop(
            0, N // (2 * L), carry=(inf_vec, inf_vec), unroll=4)(pa)
        t0 = jnp.max(jnp.maximum(mn0, mn1))
        t0v = jnp.full((L,), t0, jnp.float32)

        def pb(j, cnt):
            dv = dbuf[pl.ds(j * L, L)]
            k = dv <= t0v
            pc = plsc.all_reduce_population_count(k)[0]
            plsc.store_compressed(bufi.at[pl.ds(cnt, L)], iota + j * L,
                                  mask=k)
            return cnt + pc

        cnt = plsc.parallel_loop(
            0, N // L, carry=jnp.int32(0), unroll=8)(pb)
        n_vec = jnp.full((L,), N, jnp.int32)
        bufi[pl.ds(cnt, L)] = n_vec
        bufi[pl.ds(cnt + L, L)] = n_vec
        bufi[pl.ds(cnt + 2 * L, L)] = n_vec

        def svreg(v):
            si = bufi[pl.ds(v * L, L)]
            sk = plsc.load_gather(dbuf, [si])
            return _sort16(sk, si)

        s0k, s0i = svreg(0)
        s1k, s1i = svreg(1)
        ra = _merge16(s0k, s0i, s1k, s1i)
        rb = (inf_vec, zero_vec, inf_vec, zero_vec)

        def pc_body(u, carry):
            ra, rb = carry
            ak, ai = svreg(2 + 2 * u)
            bk, bi = svreg(3 + 2 * u)
            return _fold16(ak, ai, *ra), _fold16(bk, bi, *rb)

        nb2 = (cnt - 1) // (2 * L)
        ra, rb = lax.fori_loop(0, nb2, pc_body, (ra, rb))
        r0k, r0i, r1k, r1i = _low32(*ra, *rb)
        outv[pl.ds(m * K, L)] = r0i
        outv[pl.ds(m * K + L, L)] = r1i
        return 0

    lax.fori_loop(0, ROWS_PER_W, row_body, 0)
    pltpu.sync_copy(outv, out_hbm.at[pl.ds((b * M + row0) * K,
                                           ROWS_PER_W * K)])


@jax.jit
def _knn_sc(x1t, x2t):
    mesh = plsc.VectorSubcoreMesh(core_axis_name="c", subcore_axis_name="s")
    f = functools.partial(
        pl.kernel,
        out_type=jax.ShapeDtypeStruct((B * M * K,), jnp.int32),
        mesh=mesh,
        compiler_params=pltpu.CompilerParams(needs_layout_passes=False),
        scratch_types=[
            pltpu.VMEM((N,), jnp.float32),
            pltpu.VMEM((N,), jnp.float32),
            pltpu.VMEM((N,), jnp.float32),
            pltpu.VMEM((ROWS_PER_W,), jnp.float32),
            pltpu.VMEM((ROWS_PER_W,), jnp.float32),
            pltpu.VMEM((ROWS_PER_W,), jnp.float32),
            pltpu.VMEM((N + L,), jnp.float32),
            pltpu.VMEM((N + 4 * L,), jnp.int32),
            pltpu.VMEM((ROWS_PER_W * K,), jnp.int32),
        ],
    )(_knn_body)
    return f(x1t, x2t)


def kernel(xyz1, xyz2):
    x1t = xyz1.transpose(0, 2, 1).reshape(B * 3, N)
    x2t = xyz2.transpose(0, 2, 1).reshape(B * 3, M)
    out = _knn_sc(x1t, x2t)
    return out.reshape(B, M, K, 1)
